# Initial kernel scaffold; baseline (speedup 1.0000x reference)
import functools
import math

import jax
import jax.numpy as jnp
from jax import lax
from jax.experimental import pallas as pl
from jax.experimental.pallas import tpu as pltpu
from jax.experimental.pallas import tpu_sc as plsc

D = 128
SCALE = math.sqrt(float(D))

NC = 2
NS = 16
NW = NC * NS

IDXW = 128
K_ROWS = 4
CHUNK = K_ROWS * IDXW


def _scale_body(t_ref, o_ref):
    o_ref[...] = t_ref[...] * SCALE


def _scale_table(table):
    v, d = table.shape
    blk = 2500
    grid = (v + blk - 1) // blk
    return pl.pallas_call(
        _scale_body,
        grid=(grid,),
        in_specs=[pl.BlockSpec((blk, d), lambda i: (i, 0))],
        out_specs=pl.BlockSpec((blk, d), lambda i: (i, 0)),
        out_shape=jax.ShapeDtypeStruct((v, d), table.dtype),
    )(table)


@functools.partial(jax.jit, static_argnames=("n_rows",))
def _sc_gather(idx2d, table, n_rows):
    rows_per_w = n_rows // NW
    n_chunks = rows_per_w // K_ROWS
    mesh = plsc.VectorSubcoreMesh(
        core_axis_name="c", subcore_axis_name="s", num_cores=NC, num_subcores=NS
    )

    @functools.partial(
        pl.kernel,
        out_type=jax.ShapeDtypeStruct((n_rows * IDXW, D), jnp.float32),
        mesh=mesh,
        scratch_types=[
            pltpu.VMEM((K_ROWS, IDXW), jnp.int32),
            pltpu.VMEM((CHUNK, D), jnp.float32),
            pltpu.SemaphoreType.DMA,
        ],
    )
    def gather_kernel(idx_hbm, table_hbm, out_hbm, idx_v, rows_v, sem):
        wid = lax.axis_index("s") * NC + lax.axis_index("c")
        row0 = wid * rows_per_w

        @pl.loop(0, n_chunks)
        def _chunk(i):
            r = row0 + i * K_ROWS
            pltpu.sync_copy(idx_hbm.at[pl.ds(r, K_ROWS)], idx_v)
            copies = [
                pltpu.async_copy(
                    table_hbm.at[idx_v.at[j]],
                    rows_v.at[pl.ds(j * IDXW, IDXW)],
                    sem,
                )
                for j in range(K_ROWS)
            ]
            for c in copies:
                c.wait()
            pltpu.sync_copy(rows_v, out_hbm.at[pl.ds(r * IDXW, CHUNK)])

    return gather_kernel(idx2d, table)


def kernel(indices, table):
    b, t = indices.shape
    total = b * t
    n_rows = total // IDXW
    idx2d = indices.reshape(n_rows, IDXW)
    scaled = _scale_table(table)
    out = _sc_gather(idx2d, scaled, n_rows)
    return out.reshape(b, t, D)


# speedup vs baseline: 7.1779x; 7.1779x over previous
import functools
import math

import jax
import jax.numpy as jnp
from jax import lax
from jax.experimental import pallas as pl
from jax.experimental.pallas import tpu as pltpu
from jax.experimental.pallas import tpu_sc as plsc

D = 128
SCALE = math.sqrt(float(D))

NC = 2
NS = 16
NW = NC * NS

IDXW = 128
K_ROWS = 4
CHUNK = K_ROWS * IDXW


def _scale_body(t_ref, o_ref):
    o_ref[...] = t_ref[...] * SCALE


def _scale_table(table):
    v, d = table.shape
    blk = 2000
    grid = (v + blk - 1) // blk
    return pl.pallas_call(
        _scale_body,
        grid=(grid,),
        in_specs=[pl.BlockSpec((blk, d), lambda i: (i, 0))],
        out_specs=pl.BlockSpec((blk, d), lambda i: (i, 0)),
        out_shape=jax.ShapeDtypeStruct((v, d), table.dtype),
    )(table)


@functools.partial(jax.jit, static_argnames=("n_rows",))
def _sc_gather(idx2d, table, n_rows):
    rows_per_w = n_rows // NW
    n_chunks = rows_per_w // K_ROWS
    mesh = plsc.VectorSubcoreMesh(
        core_axis_name="c", subcore_axis_name="s", num_cores=NC, num_subcores=NS
    )

    @functools.partial(
        pl.kernel,
        out_type=jax.ShapeDtypeStruct((n_rows * IDXW, D), jnp.float32),
        mesh=mesh,
        scratch_types=[
            pltpu.VMEM((K_ROWS, IDXW), jnp.int32),
            pltpu.VMEM((CHUNK, D), jnp.float32),
            pltpu.SemaphoreType.DMA,
        ],
    )
    def gather_kernel(idx_hbm, table_hbm, out_hbm, idx_v, rows_v, sem):
        wid = lax.axis_index("s") * NC + lax.axis_index("c")
        row0 = wid * rows_per_w

        @pl.loop(0, n_chunks)
        def _chunk(i):
            r = row0 + i * K_ROWS
            pltpu.sync_copy(idx_hbm.at[pl.ds(r, K_ROWS)], idx_v)
            copies = [
                pltpu.async_copy(
                    table_hbm.at[idx_v.at[j]],
                    rows_v.at[pl.ds(j * IDXW, IDXW)],
                    sem,
                )
                for j in range(K_ROWS)
            ]
            for c in copies:
                c.wait()
            pltpu.sync_copy(rows_v, out_hbm.at[pl.ds(r * IDXW, CHUNK)])

    return gather_kernel(idx2d, table)


def kernel(indices, table):
    b, t = indices.shape
    total = b * t
    n_rows = total // IDXW
    idx2d = indices.reshape(n_rows, IDXW)
    scaled = _scale_table(table)
    out = _sc_gather(idx2d, scaled, n_rows)
    return out.reshape(b, t, D)


# speedup vs baseline: 7.9805x; 1.1118x over previous
import functools
import math

import jax
import jax.numpy as jnp
from jax import lax
from jax.experimental import pallas as pl
from jax.experimental.pallas import tpu as pltpu
from jax.experimental.pallas import tpu_sc as plsc

D = 128
SCALE = math.sqrt(float(D))

NC = 2
NS = 16
NW = NC * NS

IDXW = 128
N_BUF = 4


def _scale_body(t_ref, o_ref):
    o_ref[...] = t_ref[...] * SCALE


def _scale_table(table):
    v, d = table.shape
    blk = 2000
    grid = (v + blk - 1) // blk
    return pl.pallas_call(
        _scale_body,
        grid=(grid,),
        in_specs=[pl.BlockSpec((blk, d), lambda i: (i, 0))],
        out_specs=pl.BlockSpec((blk, d), lambda i: (i, 0)),
        out_shape=jax.ShapeDtypeStruct((v, d), table.dtype),
    )(table)


@functools.partial(jax.jit, static_argnames=("n_rows",))
def _sc_gather(idx2d, table, n_rows):
    rows_per_w = n_rows // NW
    n = rows_per_w
    mesh = plsc.VectorSubcoreMesh(
        core_axis_name="c", subcore_axis_name="s", num_cores=NC, num_subcores=NS
    )

    @functools.partial(
        pl.kernel,
        out_type=jax.ShapeDtypeStruct((n_rows * IDXW, D), jnp.float32),
        mesh=mesh,
        scratch_types=[
            pltpu.VMEM((rows_per_w, IDXW), jnp.int32),
            pltpu.VMEM((N_BUF, IDXW, D), jnp.float32),
            [pltpu.SemaphoreType.DMA] * N_BUF,
            [pltpu.SemaphoreType.DMA] * N_BUF,
        ],
    )
    def gather_kernel(idx_hbm, table_hbm, out_hbm, idx_v, rows_v, gsem, ssem):
        wid = lax.axis_index("s") * NC + lax.axis_index("c")
        row0 = wid * rows_per_w

        def fire_g(cc, b):
            pltpu.async_copy(table_hbm.at[idx_v.at[cc]], rows_v.at[b], gsem[b])

        def wait_g(cc, b):
            pltpu.make_async_copy(
                table_hbm.at[idx_v.at[cc]], rows_v.at[b], gsem[b]
            ).wait()

        def fire_s(cc, b):
            pltpu.async_copy(
                rows_v.at[b], out_hbm.at[pl.ds((row0 + cc) * IDXW, IDXW)], ssem[b]
            )

        def wait_s(b):
            pltpu.make_async_copy(
                rows_v.at[b], out_hbm.at[pl.ds(row0 * IDXW, IDXW)], ssem[b]
            ).wait()

        pltpu.sync_copy(idx_hbm.at[pl.ds(row0, rows_per_w)], idx_v)

        for b in range(3):
            fire_g(b, b)
        wait_g(0, 0)
        fire_s(0, 0)
        fire_g(3, 3)
        for cc in range(1, 4):
            wait_g(cc, cc)
            fire_s(cc, cc)
            bb = (cc + 3) % N_BUF
            wait_s(bb)
            fire_g(cc + 3, bb)

        @pl.loop(4, n - 4, step=N_BUF)
        def _group(g):
            for u in range(N_BUF):
                cc = g + u
                wait_g(cc, u)
                fire_s(cc, u)
                bb = (u + 3) % N_BUF
                wait_s(bb)
                fire_g(cc + 3, bb)

        wait_g(n - 4, 0)
        fire_s(n - 4, 0)
        wait_s(3)
        fire_g(n - 1, 3)
        for cc in range(n - 3, n):
            b = cc % N_BUF
            wait_g(cc, b)
            fire_s(cc, b)
        for b in range(N_BUF):
            wait_s(b)

    return gather_kernel(idx2d, table)


def kernel(indices, table):
    b, t = indices.shape
    total = b * t
    n_rows = total // IDXW
    idx2d = indices.reshape(n_rows, IDXW)
    scaled = _scale_table(table)
    out = _sc_gather(idx2d, scaled, n_rows)
    return out.reshape(b, t, D)
